# Initial kernel scaffold; baseline (speedup 1.0000x reference)
import jax
import jax.numpy as jnp
from jax.experimental import pallas as pl


def kernel(x):
    raise NotImplementedError("write your pallas kernel here")


# speedup vs baseline: 72.7148x; 72.7148x over previous
import functools

import jax
import jax.numpy as jnp
from jax.experimental import pallas as pl
from jax.experimental.pallas import tpu as pltpu

B, C, H, W = 8, 96, 384, 384
NCH = B * C
HW = H * W
GS = 128
G = HW // GS
K = 64
INT_MIN = -2147483648


def _sortable(x):
    u = jax.lax.bitcast_convert_type(x, jnp.int32)
    return jnp.where(u >= 0, u, u ^ jnp.int32(0x7FFFFFFF))


def _groupmax_kernel(x_ref, m_ref):
    m_ref[...] = jnp.max(x_ref[...], axis=-1)


def _select_kernel(mt_ref, idx_ref):
    s = _sortable(mt_ref[...])
    row = jax.lax.broadcasted_iota(jnp.int32, (G, NCH), 0)

    def body(i, s):
        mx = jnp.max(s, axis=0, keepdims=True)
        am = jnp.min(jnp.where(s == mx, row, jnp.int32(G)),
                     axis=0, keepdims=True)
        idx_ref[pl.ds(i, 1), :] = am
        return jnp.where(row == am, jnp.int32(INT_MIN), s)

    jax.lax.fori_loop(0, K, body, s)


def _gather_kernel(idx_sref, x_ref, c_ref):
    ch = pl.program_id(0)

    def body(i, carry):
        r = idx_sref[ch * K + i]
        c_ref[0, pl.ds(i, 1), :] = x_ref[0, pl.ds(r, 1), :]
        return carry

    jax.lax.fori_loop(0, K, body, 0)


CB = 64


def _topk_mean_kernel(c_ref, o_ref):
    x = c_ref[...]
    s = _sortable(x)

    def body(bb, acc):
        thr = acc | (jnp.int32(1) << (31 - bb))
        thr_s = thr ^ jnp.int32(INT_MIN)
        cnt = jnp.sum((s >= thr_s).astype(jnp.int32), axis=1, keepdims=True)
        return jnp.where(cnt >= K, thr, acc)

    acc = jax.lax.fori_loop(0, 32, body, jnp.zeros((CB, 1), jnp.int32))
    v_s = acc ^ jnp.int32(INT_MIN)
    vbits = jnp.where(v_s >= 0, v_s, v_s ^ jnp.int32(0x7FFFFFFF))
    v = jax.lax.bitcast_convert_type(vbits, jnp.float32)
    gt = x > v
    ngt = jnp.sum(gt.astype(jnp.float32), axis=1, keepdims=True)
    sgt = jnp.sum(jnp.where(gt, x, 0.0), axis=1, keepdims=True)
    o_ref[...] = (sgt + v * (K - ngt)) / K


@jax.jit
def kernel(x):
    x2 = x.reshape(NCH, G, GS)

    m = pl.pallas_call(
        _groupmax_kernel,
        grid=(NCH // 8,),
        in_specs=[pl.BlockSpec((8, G, GS), lambda i: (i, 0, 0))],
        out_specs=pl.BlockSpec((8, G), lambda i: (i, 0)),
        out_shape=jax.ShapeDtypeStruct((NCH, G), jnp.float32),
    )(x2)

    idx = pl.pallas_call(
        _select_kernel,
        grid=(1,),
        in_specs=[pl.BlockSpec((G, NCH), lambda i: (0, 0))],
        out_specs=pl.BlockSpec((K, NCH), lambda i: (0, 0)),
        out_shape=jax.ShapeDtypeStruct((K, NCH), jnp.int32),
    )(m.T)

    idx_flat = idx.T.reshape(-1)

    cand = pl.pallas_call(
        _gather_kernel,
        grid_spec=pltpu.PrefetchScalarGridSpec(
            num_scalar_prefetch=1,
            grid=(NCH,),
            in_specs=[pl.BlockSpec((1, G, GS), lambda ch, idx_ref: (ch, 0, 0))],
            out_specs=pl.BlockSpec((1, K, GS), lambda ch, idx_ref: (ch, 0, 0)),
        ),
        out_shape=jax.ShapeDtypeStruct((NCH, K, GS), jnp.float32),
    )(idx_flat, x2)

    out = pl.pallas_call(
        _topk_mean_kernel,
        grid=(NCH // CB,),
        in_specs=[pl.BlockSpec((CB, K * GS), lambda i: (i, 0))],
        out_specs=pl.BlockSpec((CB, 1), lambda i: (i, 0)),
        out_shape=jax.ShapeDtypeStruct((NCH, 1), jnp.float32),
    )(cand.reshape(NCH, K * GS))

    return out.reshape(B, C, 1, 1)


# speedup vs baseline: 78.9541x; 1.0858x over previous
import functools

import jax
import jax.numpy as jnp
from jax import lax
from jax.experimental import pallas as pl
from jax.experimental.pallas import tpu as pltpu
from jax.experimental.pallas import tpu_sc as plsc

B, C, H, W = 8, 96, 384, 384
NCH = B * C
HW = H * W
GS = 128
G = HW // GS
K = 64
INT_MIN = -2147483648


def _sortable(x):
    u = jax.lax.bitcast_convert_type(x, jnp.int32)
    return jnp.where(u >= 0, u, u ^ jnp.int32(0x7FFFFFFF))


def _groupmax_kernel(x_ref, m_ref):
    m_ref[...] = jnp.max(x_ref[...], axis=-1)


def _select_kernel(mt_ref, idx_ref):
    s = _sortable(mt_ref[...])
    row = jax.lax.broadcasted_iota(jnp.int32, (G, NCH), 0)

    col = jax.lax.broadcasted_iota(jnp.int32, (1, NCH), 1)

    def body(i, s):
        mx = jnp.max(s, axis=0, keepdims=True)
        am = jnp.min(jnp.where(s == mx, row, jnp.int32(G)),
                     axis=0, keepdims=True)
        idx_ref[pl.ds(i, 1), :] = am + col * G
        return jnp.where(row == am, jnp.int32(INT_MIN), s)

    jax.lax.fori_loop(0, K, body, s)


NW = 32
ROWS = NCH * K
BPW = ROWS // NW
CH_ROWS = 512


def _sc_gather_kernel(table_hbm, idx_hbm, out_hbm, idx_v, rows_v, sem):
    wid = lax.axis_index("s") * 2 + lax.axis_index("c")
    base = wid * BPW

    def body(t, carry):
        off = base + t * CH_ROWS
        pltpu.sync_copy(idx_hbm.at[pl.ds(off, CH_ROWS)], idx_v)
        pltpu.async_copy(table_hbm.at[idx_v], rows_v, sem).wait()
        pltpu.sync_copy(rows_v, out_hbm.at[pl.ds(off, CH_ROWS)])
        return carry

    jax.lax.fori_loop(0, BPW // CH_ROWS, body, 0)


_sc_gather = functools.partial(
    pl.kernel,
    mesh=plsc.VectorSubcoreMesh(core_axis_name="c", subcore_axis_name="s"),
    out_type=jax.ShapeDtypeStruct((ROWS, GS), jnp.float32),
    scratch_types=[
        pltpu.VMEM((CH_ROWS,), jnp.int32),
        pltpu.VMEM((CH_ROWS, GS), jnp.float32),
        pltpu.SemaphoreType.DMA,
    ],
)(_sc_gather_kernel)


CB = 64


def _topk_mean_kernel(c_ref, o_ref):
    x = c_ref[...]
    s = _sortable(x)

    def body(bb, acc):
        thr = acc | (jnp.int32(1) << (31 - bb))
        thr_s = thr ^ jnp.int32(INT_MIN)
        cnt = jnp.sum((s >= thr_s).astype(jnp.int32), axis=1, keepdims=True)
        return jnp.where(cnt >= K, thr, acc)

    acc = jax.lax.fori_loop(0, 32, body, jnp.zeros((CB, 1), jnp.int32))
    v_s = acc ^ jnp.int32(INT_MIN)
    vbits = jnp.where(v_s >= 0, v_s, v_s ^ jnp.int32(0x7FFFFFFF))
    v = jax.lax.bitcast_convert_type(vbits, jnp.float32)
    gt = x > v
    ngt = jnp.sum(gt.astype(jnp.float32), axis=1, keepdims=True)
    sgt = jnp.sum(jnp.where(gt, x, 0.0), axis=1, keepdims=True)
    o_ref[...] = (sgt + v * (K - ngt)) / K


@jax.jit
def kernel(x):
    x2 = x.reshape(NCH, G, GS)

    m = pl.pallas_call(
        _groupmax_kernel,
        grid=(NCH // 8,),
        in_specs=[pl.BlockSpec((8, G, GS), lambda i: (i, 0, 0))],
        out_specs=pl.BlockSpec((8, G), lambda i: (i, 0)),
        out_shape=jax.ShapeDtypeStruct((NCH, G), jnp.float32),
    )(x2)

    idx = pl.pallas_call(
        _select_kernel,
        grid=(1,),
        in_specs=[pl.BlockSpec((G, NCH), lambda i: (0, 0))],
        out_specs=pl.BlockSpec((K, NCH), lambda i: (0, 0)),
        out_shape=jax.ShapeDtypeStruct((K, NCH), jnp.int32),
    )(m.T)

    idx_flat = idx.T.reshape(-1)

    cand = _sc_gather(x2.reshape(NCH * G, GS), idx_flat)

    out = pl.pallas_call(
        _topk_mean_kernel,
        grid=(NCH // CB,),
        in_specs=[pl.BlockSpec((CB, K * GS), lambda i: (i, 0))],
        out_specs=pl.BlockSpec((CB, 1), lambda i: (i, 0)),
        out_shape=jax.ShapeDtypeStruct((NCH, 1), jnp.float32),
    )(cand.reshape(NCH, K * GS))

    return out.reshape(B, C, 1, 1)


# speedup vs baseline: 81.0982x; 1.0272x over previous
import functools

import jax
import jax.numpy as jnp
from jax import lax
from jax.experimental import pallas as pl
from jax.experimental.pallas import tpu as pltpu
from jax.experimental.pallas import tpu_sc as plsc

B, C, H, W = 8, 96, 384, 384
NCH = B * C
HW = H * W
GS = 128
G = HW // GS
K = 64
INT_MIN = -2147483648


def _sortable(x):
    u = jax.lax.bitcast_convert_type(x, jnp.int32)
    return jnp.where(u >= 0, u, u ^ jnp.int32(0x7FFFFFFF))


def _groupmax_kernel(x_ref, m_ref):
    m_ref[...] = jnp.max(x_ref[...], axis=-1)


def _make_select_kernel(g, row_id_fn):
    def _select_kernel(mt_ref, idx_ref):
        s = _sortable(mt_ref[...])
        row = jax.lax.broadcasted_iota(jnp.int32, (g, NCH), 0)
        col = jax.lax.broadcasted_iota(jnp.int32, (1, NCH), 1)

        def body(i, s):
            mx = jnp.max(s, axis=0, keepdims=True)
            am = jnp.min(jnp.where(s == mx, row, jnp.int32(g)),
                         axis=0, keepdims=True)
            idx_ref[pl.ds(i, 1), :] = row_id_fn(am, col)
            return jnp.where(row == am, jnp.int32(INT_MIN), s)

        jax.lax.fori_loop(0, K, body, s)

    return _select_kernel


def _submax_kernel(x_ref, m_ref):
    m_ref[...] = jnp.max(x_ref[...], axis=1)


NW = 32
ROWS = NCH * K
BPW = ROWS // NW
CH_ROWS = 512


def _make_sc_gather(width):
    def _sc_gather_kernel(table_hbm, idx_hbm, out_hbm, idx_v, rows_v, sem):
        wid = lax.axis_index("s") * 2 + lax.axis_index("c")
        base = wid * BPW

        def body(t, carry):
            off = base + t * CH_ROWS
            pltpu.sync_copy(idx_hbm.at[pl.ds(off, CH_ROWS)], idx_v)
            pltpu.async_copy(table_hbm.at[idx_v], rows_v, sem).wait()
            pltpu.sync_copy(rows_v, out_hbm.at[pl.ds(off, CH_ROWS)])
            return carry

        jax.lax.fori_loop(0, BPW // CH_ROWS, body, 0)

    return functools.partial(
        pl.kernel,
        mesh=plsc.VectorSubcoreMesh(core_axis_name="c", subcore_axis_name="s"),
        out_type=jax.ShapeDtypeStruct((ROWS, width), jnp.float32),
        scratch_types=[
            pltpu.VMEM((CH_ROWS,), jnp.int32),
            pltpu.VMEM((CH_ROWS, width), jnp.float32),
            pltpu.SemaphoreType.DMA,
        ],
    )(_sc_gather_kernel)


CPW = NCH // NW


def _sc_gather2_kernel(cand_hbm, idx_hbm, out_hbm, g_v, blk_v, out_v):
    wid = lax.axis_index("s") * 2 + lax.axis_index("c")

    def chan_body(cc, carry):
        c = wid * CPW + cc
        pltpu.sync_copy(cand_hbm.at[pl.ds(c * K * GS, K * GS)], blk_v)
        pltpu.sync_copy(idx_hbm.at[pl.ds(c, 1)], g_v)

        def jb(jj, carry2):
            gs16 = g_v[0, pl.ds(jj * 16, 16)]
            base16 = (gs16 >> 7) * 2048 + (gs16 & 127)

            for i in range(16):
                v = plsc.load_gather(blk_v, [base16 + i * 128])
                out_v[0, pl.ds(jj * 256 + i * 16, 16)] = v
            return carry2

        jax.lax.fori_loop(0, 4, jb, 0)
        pltpu.sync_copy(out_v, out_hbm.at[pl.ds(c, 1)])
        return carry

    jax.lax.fori_loop(0, CPW, chan_body, 0)


def _make_sc_gather2():
    return functools.partial(
        pl.kernel,
        mesh=plsc.VectorSubcoreMesh(core_axis_name="c", subcore_axis_name="s"),
        out_type=jax.ShapeDtypeStruct((NCH, K * 16), jnp.float32),
        compiler_params=pltpu.CompilerParams(needs_layout_passes=False),
        scratch_types=[
            pltpu.VMEM((1, K), jnp.int32),
            pltpu.VMEM((K * GS,), jnp.float32),
            pltpu.VMEM((1, K * 16), jnp.float32),
        ],
    )(_sc_gather2_kernel)


def _topk_mean_kernel(c_ref, o_ref):
    x = c_ref[...]
    cb = x.shape[0]
    s = _sortable(x)

    def body(bb, acc):
        thr = acc | (jnp.int32(1) << (31 - bb))
        thr_s = thr ^ jnp.int32(INT_MIN)
        cnt = jnp.sum((s >= thr_s).astype(jnp.int32), axis=1, keepdims=True)
        return jnp.where(cnt >= K, thr, acc)

    acc = jax.lax.fori_loop(0, 32, body, jnp.zeros((cb, 1), jnp.int32))
    v_s = acc ^ jnp.int32(INT_MIN)
    vbits = jnp.where(v_s >= 0, v_s, v_s ^ jnp.int32(0x7FFFFFFF))
    v = jax.lax.bitcast_convert_type(vbits, jnp.float32)
    gt = x > v
    ngt = jnp.sum(gt.astype(jnp.float32), axis=1, keepdims=True)
    sgt = jnp.sum(jnp.where(gt, x, 0.0), axis=1, keepdims=True)
    o_ref[...] = (sgt + v * (K - ngt)) / K


@jax.jit
def kernel(x):
    x2 = x.reshape(NCH, G, GS)

    m = pl.pallas_call(
        _groupmax_kernel,
        grid=(NCH // 8,),
        in_specs=[pl.BlockSpec((8, G, GS), lambda i: (i, 0, 0))],
        out_specs=pl.BlockSpec((8, G), lambda i: (i, 0)),
        out_shape=jax.ShapeDtypeStruct((NCH, G), jnp.float32),
    )(x2)

    idx = pl.pallas_call(
        _make_select_kernel(G, lambda am, col: am + col * G),
        grid=(1,),
        in_specs=[pl.BlockSpec((G, NCH), lambda i: (0, 0))],
        out_specs=pl.BlockSpec((K, NCH), lambda i: (0, 0)),
        out_shape=jax.ShapeDtypeStruct((K, NCH), jnp.int32),
    )(m.T)

    idx_flat = idx.T.reshape(-1)

    cand = _make_sc_gather(GS)(x2.reshape(NCH * G, GS), idx_flat)

    G2 = 512
    m2 = pl.pallas_call(
        _submax_kernel,
        grid=(NCH * 4 // 128,),
        in_specs=[pl.BlockSpec((128, 16, GS), lambda i: (i, 0, 0))],
        out_specs=pl.BlockSpec((128, GS), lambda i: (i, 0)),
        out_shape=jax.ShapeDtypeStruct((NCH * 4, GS), jnp.float32),
    )(cand.reshape(NCH * 4, 16, GS))

    idx2 = pl.pallas_call(
        _make_select_kernel(G2, lambda am, col: am),
        grid=(1,),
        in_specs=[pl.BlockSpec((G2, NCH), lambda i: (0, 0))],
        out_specs=pl.BlockSpec((K, NCH), lambda i: (0, 0)),
        out_shape=jax.ShapeDtypeStruct((K, NCH), jnp.int32),
    )(m2.reshape(NCH, G2).T)

    cand2 = _make_sc_gather2()(cand.reshape(-1), idx2.T)

    out = pl.pallas_call(
        _topk_mean_kernel,
        grid=(1,),
        in_specs=[pl.BlockSpec((NCH, K * 16), lambda i: (0, 0))],
        out_specs=pl.BlockSpec((NCH, 1), lambda i: (0, 0)),
        out_shape=jax.ShapeDtypeStruct((NCH, 1), jnp.float32),
    )(cand2)

    return out.reshape(B, C, 1, 1)


# speedup vs baseline: 83.5198x; 1.0299x over previous
import functools

import jax
import jax.numpy as jnp
from jax import lax
from jax.experimental import pallas as pl
from jax.experimental.pallas import tpu as pltpu
from jax.experimental.pallas import tpu_sc as plsc

B, C, H, W = 8, 96, 384, 384
NCH = B * C
HW = H * W
GS = 128
G = HW // GS
K = 64
INT_MIN = -2147483648


def _sortable(x):
    u = jax.lax.bitcast_convert_type(x, jnp.int32)
    return jnp.where(u >= 0, u, u ^ jnp.int32(0x7FFFFFFF))


def _groupmax_kernel(x_ref, m_ref):
    m_ref[...] = jnp.max(x_ref[...], axis=-1)


def _make_select_kernel(g, row_id_fn):
    def _select_kernel(mt_ref, idx_ref):
        s = _sortable(mt_ref[...])
        row = jax.lax.broadcasted_iota(jnp.int32, (g, NCH), 0)
        col = jax.lax.broadcasted_iota(jnp.int32, (1, NCH), 1)

        def body(i, s):
            mx = jnp.max(s, axis=0, keepdims=True)
            am = jnp.min(jnp.where(s == mx, row, jnp.int32(g)),
                         axis=0, keepdims=True)
            idx_ref[pl.ds(i, 1), :] = row_id_fn(am, col)
            return jnp.where(row == am, jnp.int32(INT_MIN), s)

        jax.lax.fori_loop(0, K, body, s)

    return _select_kernel


def _submax_kernel(x_ref, m_ref):
    m_ref[...] = jnp.max(x_ref[...], axis=1)


NW = 32
ROWS = NCH * K
BPW = ROWS // NW
CH_ROWS = 512


def _make_sc_gather(width):
    def _sc_gather_kernel(table_hbm, idx_hbm, out_hbm, idx_v, rows_v, sem):
        wid = lax.axis_index("s") * 2 + lax.axis_index("c")
        base = wid * BPW

        def body(t, carry):
            off = base + t * CH_ROWS
            pltpu.sync_copy(idx_hbm.at[pl.ds(off, CH_ROWS)], idx_v)
            pltpu.async_copy(table_hbm.at[idx_v], rows_v, sem).wait()
            pltpu.sync_copy(rows_v, out_hbm.at[pl.ds(off, CH_ROWS)])
            return carry

        jax.lax.fori_loop(0, BPW // CH_ROWS, body, 0)

    return functools.partial(
        pl.kernel,
        mesh=plsc.VectorSubcoreMesh(core_axis_name="c", subcore_axis_name="s"),
        out_type=jax.ShapeDtypeStruct((ROWS, width), jnp.float32),
        scratch_types=[
            pltpu.VMEM((CH_ROWS,), jnp.int32),
            pltpu.VMEM((CH_ROWS, width), jnp.float32),
            pltpu.SemaphoreType.DMA,
        ],
    )(_sc_gather_kernel)


CPW = NCH // NW


def _sc_gather2_kernel(cand_hbm, idx_hbm, out_hbm, g_v, blk_v, out_v):
    wid = lax.axis_index("s") * 2 + lax.axis_index("c")

    def chan_body(cc, carry):
        c = wid * CPW + cc
        pltpu.sync_copy(cand_hbm.at[pl.ds(c * K * GS, K * GS)], blk_v)
        pltpu.sync_copy(idx_hbm.at[pl.ds(c, 1)], g_v)

        def jb(jj, carry2):
            gs16 = g_v[0, pl.ds(jj * 16, 16)]
            base16 = (gs16 >> 7) * 2048 + (gs16 & 127)

            for i in range(16):
                v = plsc.load_gather(blk_v, [base16 + i * 128])
                out_v[0, pl.ds(jj * 256 + i * 16, 16)] = v
            return carry2

        jax.lax.fori_loop(0, 4, jb, 0)
        pltpu.sync_copy(out_v, out_hbm.at[pl.ds(c, 1)])
        return carry

    jax.lax.fori_loop(0, CPW, chan_body, 0)


def _make_sc_gather2():
    return functools.partial(
        pl.kernel,
        mesh=plsc.VectorSubcoreMesh(core_axis_name="c", subcore_axis_name="s"),
        out_type=jax.ShapeDtypeStruct((NCH, K * 16), jnp.float32),
        compiler_params=pltpu.CompilerParams(needs_layout_passes=False),
        scratch_types=[
            pltpu.VMEM((1, K), jnp.int32),
            pltpu.VMEM((K * GS,), jnp.float32),
            pltpu.VMEM((1, K * 16), jnp.float32),
        ],
    )(_sc_gather2_kernel)


def _topk_mean_kernel(c_ref, o_ref):
    x = c_ref[...]
    cb = x.shape[0]
    s = _sortable(x)

    def body(bb, acc):
        thr = acc | (jnp.int32(1) << (31 - bb))
        thr_s = thr ^ jnp.int32(INT_MIN)
        cnt = jnp.sum((s >= thr_s).astype(jnp.int32), axis=1, keepdims=True)
        return jnp.where(cnt >= K, thr, acc)

    acc = jax.lax.fori_loop(0, 32, body, jnp.zeros((cb, 1), jnp.int32))
    v_s = acc ^ jnp.int32(INT_MIN)
    vbits = jnp.where(v_s >= 0, v_s, v_s ^ jnp.int32(0x7FFFFFFF))
    v = jax.lax.bitcast_convert_type(vbits, jnp.float32)
    gt = x > v
    ngt = jnp.sum(gt.astype(jnp.float32), axis=1, keepdims=True)
    sgt = jnp.sum(jnp.where(gt, x, 0.0), axis=1, keepdims=True)
    o_ref[...] = (sgt + v * (K - ngt)) / K


@jax.jit
def kernel(x):
    x2 = x.reshape(NCH, G, GS)

    m = pl.pallas_call(
        _groupmax_kernel,
        grid=(NCH // 32,),
        in_specs=[pl.BlockSpec((32, G, GS), lambda i: (i, 0, 0))],
        out_specs=pl.BlockSpec((32, G), lambda i: (i, 0)),
        out_shape=jax.ShapeDtypeStruct((NCH, G), jnp.float32),
    )(x2)

    idx = pl.pallas_call(
        _make_select_kernel(G, lambda am, col: am + col * G),
        grid=(1,),
        in_specs=[pl.BlockSpec((G, NCH), lambda i: (0, 0))],
        out_specs=pl.BlockSpec((K, NCH), lambda i: (0, 0)),
        out_shape=jax.ShapeDtypeStruct((K, NCH), jnp.int32),
    )(m.T)

    idx_flat = idx.T.reshape(-1)

    cand = _make_sc_gather(GS)(x2.reshape(NCH * G, GS), idx_flat)

    G2 = 512
    m2 = pl.pallas_call(
        _submax_kernel,
        grid=(NCH * 4 // 128,),
        in_specs=[pl.BlockSpec((128, 16, GS), lambda i: (i, 0, 0))],
        out_specs=pl.BlockSpec((128, GS), lambda i: (i, 0)),
        out_shape=jax.ShapeDtypeStruct((NCH * 4, GS), jnp.float32),
    )(cand.reshape(NCH * 4, 16, GS))

    idx2 = pl.pallas_call(
        _make_select_kernel(G2, lambda am, col: am),
        grid=(1,),
        in_specs=[pl.BlockSpec((G2, NCH), lambda i: (0, 0))],
        out_specs=pl.BlockSpec((K, NCH), lambda i: (0, 0)),
        out_shape=jax.ShapeDtypeStruct((K, NCH), jnp.int32),
    )(m2.reshape(NCH, G2).T)

    cand2 = _make_sc_gather2()(cand.reshape(-1), idx2.T)

    out = pl.pallas_call(
        _topk_mean_kernel,
        grid=(1,),
        in_specs=[pl.BlockSpec((NCH, K * 16), lambda i: (0, 0))],
        out_specs=pl.BlockSpec((NCH, 1), lambda i: (0, 0)),
        out_shape=jax.ShapeDtypeStruct((NCH, 1), jnp.float32),
    )(cand2)

    return out.reshape(B, C, 1, 1)


# speedup vs baseline: 83.5474x; 1.0003x over previous
import functools

import jax
import jax.numpy as jnp
from jax import lax
from jax.experimental import pallas as pl
from jax.experimental.pallas import tpu as pltpu
from jax.experimental.pallas import tpu_sc as plsc

B, C, H, W = 8, 96, 384, 384
NCH = B * C
HW = H * W
GS = 128
G = HW // GS
K = 64
INT_MIN = -2147483648


def _sortable(x):
    u = jax.lax.bitcast_convert_type(x, jnp.int32)
    return jnp.where(u >= 0, u, u ^ jnp.int32(0x7FFFFFFF))


def _groupmax_kernel(x_ref, m_ref):
    m_ref[...] = jnp.max(x_ref[...], axis=-1)


def _make_select_kernel(g, row_id_fn):
    def _select_kernel(mt_ref, idx_ref):
        s = _sortable(mt_ref[...])
        row = jax.lax.broadcasted_iota(jnp.int32, (g, NCH), 0)
        col = jax.lax.broadcasted_iota(jnp.int32, (1, NCH), 1)

        def body(i, s):
            mx = jnp.max(s, axis=0, keepdims=True)
            am = jnp.min(jnp.where(s == mx, row, jnp.int32(g)),
                         axis=0, keepdims=True)
            idx_ref[pl.ds(i, 1), :] = row_id_fn(am, col)
            return jnp.where(row == am, jnp.int32(INT_MIN), s)

        jax.lax.fori_loop(0, K, body, s)

    return _select_kernel


def _submax_kernel(x_ref, m_ref):
    m_ref[...] = jnp.max(x_ref[...], axis=1)


NW = 32
ROWS = NCH * K
BPW = ROWS // NW
CH_ROWS = 512


def _make_sc_gather(width):
    def _sc_gather_kernel(table_hbm, idx_hbm, out_hbm, idx_v, rows_v, sem):
        wid = lax.axis_index("s") * 2 + lax.axis_index("c")
        base = wid * BPW

        def body(t, carry):
            off = base + t * CH_ROWS
            pltpu.sync_copy(idx_hbm.at[pl.ds(off, CH_ROWS)], idx_v)
            pltpu.async_copy(table_hbm.at[idx_v], rows_v, sem).wait()
            pltpu.sync_copy(rows_v, out_hbm.at[pl.ds(off, CH_ROWS)])
            return carry

        jax.lax.fori_loop(0, BPW // CH_ROWS, body, 0)

    return functools.partial(
        pl.kernel,
        mesh=plsc.VectorSubcoreMesh(core_axis_name="c", subcore_axis_name="s"),
        out_type=jax.ShapeDtypeStruct((ROWS, width), jnp.float32),
        scratch_types=[
            pltpu.VMEM((CH_ROWS,), jnp.int32),
            pltpu.VMEM((CH_ROWS, width), jnp.float32),
            pltpu.SemaphoreType.DMA,
        ],
    )(_sc_gather_kernel)


CPW = NCH // NW


def _sc_gather2_kernel(cand_hbm, idx_hbm, out_hbm, g_v, blk_v, out_v):
    wid = lax.axis_index("s") * 2 + lax.axis_index("c")

    def chan_body(cc, carry):
        c = wid * CPW + cc
        pltpu.sync_copy(cand_hbm.at[pl.ds(c * K * GS, K * GS)], blk_v)
        pltpu.sync_copy(idx_hbm.at[pl.ds(c, 1)], g_v)

        def jb(jj, carry2):
            gs16 = g_v[0, pl.ds(jj * 16, 16)]
            base16 = (gs16 >> 7) * 2048 + (gs16 & 127)

            for i in range(16):
                v = plsc.load_gather(blk_v, [base16 + i * 128])
                out_v[0, pl.ds(jj * 256 + i * 16, 16)] = v
            return carry2

        jax.lax.fori_loop(0, 4, jb, 0)
        pltpu.sync_copy(out_v, out_hbm.at[pl.ds(c, 1)])
        return carry

    jax.lax.fori_loop(0, CPW, chan_body, 0)


def _make_sc_gather2():
    return functools.partial(
        pl.kernel,
        mesh=plsc.VectorSubcoreMesh(core_axis_name="c", subcore_axis_name="s"),
        out_type=jax.ShapeDtypeStruct((NCH, K * 16), jnp.float32),
        compiler_params=pltpu.CompilerParams(needs_layout_passes=False),
        scratch_types=[
            pltpu.VMEM((1, K), jnp.int32),
            pltpu.VMEM((K * GS,), jnp.float32),
            pltpu.VMEM((1, K * 16), jnp.float32),
        ],
    )(_sc_gather2_kernel)


def _topk_mean_kernel(c_ref, o_ref):
    x = c_ref[...]
    cb = x.shape[0]
    s = _sortable(x)

    def body(bb, acc):
        thr = acc | (jnp.int32(1) << (31 - bb))
        thr_s = thr ^ jnp.int32(INT_MIN)
        cnt = jnp.sum((s >= thr_s).astype(jnp.int32), axis=1, keepdims=True)
        return jnp.where(cnt >= K, thr, acc)

    acc = jax.lax.fori_loop(0, 32, body, jnp.zeros((cb, 1), jnp.int32))
    v_s = acc ^ jnp.int32(INT_MIN)
    vbits = jnp.where(v_s >= 0, v_s, v_s ^ jnp.int32(0x7FFFFFFF))
    v = jax.lax.bitcast_convert_type(vbits, jnp.float32)
    gt = x > v
    ngt = jnp.sum(gt.astype(jnp.float32), axis=1, keepdims=True)
    sgt = jnp.sum(jnp.where(gt, x, 0.0), axis=1, keepdims=True)
    o_ref[...] = (sgt + v * (K - ngt)) / K


@jax.jit
def kernel(x):
    x2 = x.reshape(NCH, G, GS)

    m = pl.pallas_call(
        _groupmax_kernel,
        grid=(NCH // 48,),
        in_specs=[pl.BlockSpec((48, G, GS), lambda i: (i, 0, 0))],
        out_specs=pl.BlockSpec((48, G), lambda i: (i, 0)),
        out_shape=jax.ShapeDtypeStruct((NCH, G), jnp.float32),
    )(x2)

    idx = pl.pallas_call(
        _make_select_kernel(G, lambda am, col: am + col * G),
        grid=(1,),
        in_specs=[pl.BlockSpec((G, NCH), lambda i: (0, 0))],
        out_specs=pl.BlockSpec((K, NCH), lambda i: (0, 0)),
        out_shape=jax.ShapeDtypeStruct((K, NCH), jnp.int32),
    )(m.T)

    idx_flat = idx.T.reshape(-1)

    cand = _make_sc_gather(GS)(x2.reshape(NCH * G, GS), idx_flat)

    G2 = 512
    m2 = pl.pallas_call(
        _submax_kernel,
        grid=(NCH * 4 // 128,),
        in_specs=[pl.BlockSpec((128, 16, GS), lambda i: (i, 0, 0))],
        out_specs=pl.BlockSpec((128, GS), lambda i: (i, 0)),
        out_shape=jax.ShapeDtypeStruct((NCH * 4, GS), jnp.float32),
    )(cand.reshape(NCH * 4, 16, GS))

    idx2 = pl.pallas_call(
        _make_select_kernel(G2, lambda am, col: am),
        grid=(1,),
        in_specs=[pl.BlockSpec((G2, NCH), lambda i: (0, 0))],
        out_specs=pl.BlockSpec((K, NCH), lambda i: (0, 0)),
        out_shape=jax.ShapeDtypeStruct((K, NCH), jnp.int32),
    )(m2.reshape(NCH, G2).T)

    cand2 = _make_sc_gather2()(cand.reshape(-1), idx2.T)

    out = pl.pallas_call(
        _topk_mean_kernel,
        grid=(1,),
        in_specs=[pl.BlockSpec((NCH, K * 16), lambda i: (0, 0))],
        out_specs=pl.BlockSpec((NCH, 1), lambda i: (0, 0)),
        out_shape=jax.ShapeDtypeStruct((NCH, 1), jnp.float32),
    )(cand2)

    return out.reshape(B, C, 1, 1)
